# Initial kernel scaffold; baseline (speedup 1.0000x reference)
import jax
import jax.numpy as jnp
from jax.experimental import pallas as pl


def kernel(x, edge_index, W_self0, W_neigh0, b0, W_self1, W_neigh1, b1, W_self2, W_neigh2, b2, gamma0, beta0, gamma1, beta1):
    raise NotImplementedError("write your pallas kernel here")


# speedup vs baseline: 6.8142x; 6.8142x over previous
import functools

import jax
import jax.numpy as jnp
from jax import lax
from jax.experimental import pallas as pl
from jax.experimental.pallas import tpu as pltpu
from jax.experimental.pallas import tpu_sc as plsc

NC = 2
NS = 16
K = 80


def _make_sc_aggregate(N, F, CH, with_deg):
    NW = NC * NS
    chunks_per_tec = CH // NW
    rows_per_tile = (N // NS) // 8 * 8
    tail_rows = N - NS * rows_per_tile
    tail_base = NS * rows_per_tile
    mesh = plsc.VectorSubcoreMesh(core_axis_name="c", subcore_axis_name="s")

    out_type = [jax.ShapeDtypeStruct((NC, N, F), jnp.float32)]
    scratch = [
        pltpu.VMEM((chunks_per_tec, K), jnp.int32),
        pltpu.VMEM((chunks_per_tec, K), jnp.int32),
    ]
    scratch += [
        pltpu.VMEM((K, F), jnp.float32),
        pltpu.VMEM_SHARED((N, F), jnp.float32),
        pltpu.SemaphoreType.DMA,
    ]
    if with_deg:
        out_type.append(jax.ShapeDtypeStruct((NC, N), jnp.float32))
        scratch += [
            pltpu.VMEM((K,), jnp.float32),
            pltpu.VMEM_SHARED((N,), jnp.float32),
        ]

    if with_deg:
        def body(t_hbm, src_hbm, dst_hbm, z2_hbm, z1_hbm, out_hbm, deg_hbm,
                 src_v, dst_v, rows_v, acc, sem, ones_v, dacc):
            _agg_body(t_hbm, src_hbm, dst_hbm, z2_hbm, out_hbm,
                      src_v, dst_v, rows_v, acc, sem,
                      z1_hbm=z1_hbm, deg_hbm=deg_hbm, ones_v=ones_v,
                      dacc=dacc)
    else:
        def body(t_hbm, src_hbm, dst_hbm, z2_hbm, out_hbm,
                 src_v, dst_v, rows_v, acc, sem):
            _agg_body(t_hbm, src_hbm, dst_hbm, z2_hbm, out_hbm,
                      src_v, dst_v, rows_v, acc, sem)

    def _agg_body(t_hbm, src_hbm, dst_hbm, z2_hbm, out_hbm,
                  src_v, dst_v, rows_v, acc, sem,
                  z1_hbm=None, deg_hbm=None, ones_v=None, dacc=None):
        c = lax.axis_index("c")
        s = lax.axis_index("s")
        w = c * NS + s
        pltpu.sync_copy(z2_hbm.at[pl.ds(s * rows_per_tile, rows_per_tile)],
                        acc.at[pl.ds(s * rows_per_tile, rows_per_tile)])
        if tail_rows:
            @pl.when(s == NS - 1)
            def _():
                pltpu.sync_copy(z2_hbm.at[pl.ds(tail_base, tail_rows)],
                                acc.at[pl.ds(tail_base, tail_rows)])
        if dacc is not None:
            @pl.when(s == 0)
            def _():
                pltpu.sync_copy(z1_hbm, dacc)
            for j in range(K // 16):
                ones_v[pl.ds(j * 16, 16)] = jnp.ones((16,), jnp.float32)
        pltpu.sync_copy(src_hbm.at[w], src_v)
        pltpu.sync_copy(dst_hbm.at[w], dst_v)
        plsc.subcore_barrier()

        def step(i, carry):
            pltpu.async_copy(t_hbm.at[src_v.at[i]], rows_v, sem).wait()
            pltpu.sync_copy(rows_v, acc.at[dst_v.at[i]], add=True)
            if dacc is not None:
                pltpu.sync_copy(ones_v, dacc.at[dst_v.at[i]], add=True)
            return carry

        lax.fori_loop(0, chunks_per_tec, step, 0)
        plsc.subcore_barrier()
        pltpu.sync_copy(acc.at[pl.ds(s * rows_per_tile, rows_per_tile)],
                        out_hbm.at[c, pl.ds(s * rows_per_tile, rows_per_tile)])
        if tail_rows:
            @pl.when(s == NS - 1)
            def _():
                pltpu.sync_copy(acc.at[pl.ds(tail_base, tail_rows)],
                                out_hbm.at[c, pl.ds(tail_base, tail_rows)])
        if dacc is not None:
            @pl.when(s == 0)
            def _():
                pltpu.sync_copy(dacc, deg_hbm.at[c])

    return pl.kernel(body, out_type=tuple(out_type), mesh=mesh,
                     scratch_types=scratch)


def _tc_layer(h, parts, deg_t, W_self, W_neigh, b, gamma, beta, apply_ln):
    N, Fin = h.shape
    Fout = W_self.shape[1]
    BLK = 1000
    grid = (N // BLK,)

    def body(h_ref, p_ref, d_ref, ws_ref, wn_ref, b_ref, g_ref, be_ref, o_ref):
        hb = h_ref[...]
        agg = p_ref[0] + p_ref[1]
        deg = jnp.sum(d_ref[...], axis=1, keepdims=True)
        inv = 1.0 / jnp.maximum(deg, 1.0)
        hn = agg * inv
        z = jnp.dot(hb, ws_ref[...], preferred_element_type=jnp.float32)
        z = z + jnp.dot(hn, wn_ref[...], preferred_element_type=jnp.float32)
        z = z + b_ref[...]
        if apply_ln:
            mu = jnp.mean(z, axis=-1, keepdims=True)
            var = jnp.mean((z - mu) ** 2, axis=-1, keepdims=True)
            z = (z - mu) * lax.rsqrt(var + 1e-5) * g_ref[...] + be_ref[...]
            z = jnp.maximum(z, 0.0)
        o_ref[...] = z

    return pl.pallas_call(
        body,
        grid=grid,
        in_specs=[
            pl.BlockSpec((BLK, Fin), lambda i: (i, 0)),
            pl.BlockSpec((NC, BLK, Fin), lambda i: (0, i, 0)),
            pl.BlockSpec((BLK, NC), lambda i: (i, 0)),
            pl.BlockSpec((Fin, Fout), lambda i: (0, 0)),
            pl.BlockSpec((Fin, Fout), lambda i: (0, 0)),
            pl.BlockSpec((1, Fout), lambda i: (0, 0)),
            pl.BlockSpec((1, Fout), lambda i: (0, 0)),
            pl.BlockSpec((1, Fout), lambda i: (0, 0)),
        ],
        out_specs=pl.BlockSpec((BLK, Fout), lambda i: (i, 0)),
        out_shape=jax.ShapeDtypeStruct((N, Fout), jnp.float32),
    )(h, parts, deg_t, W_self, W_neigh, b, gamma, beta)


def kernel(x, edge_index, W_self0, W_neigh0, b0, W_self1, W_neigh1, b1,
           W_self2, W_neigh2, b2, gamma0, beta0, gamma1, beta1):
    N, D = x.shape
    E = edge_index.shape[1]
    H = W_self0.shape[1]
    C = W_self2.shape[1]
    CH = E // K
    assert E % K == 0 and CH % (NC * NS) == 0 and N % NS == 0

    NW = NC * NS
    src2d = edge_index[0].astype(jnp.int32).reshape(NW, CH // NW, K)
    dst2d = edge_index[1].astype(jnp.int32).reshape(NW, CH // NW, K)
    z2 = jnp.zeros((N, D), jnp.float32)
    z1 = jnp.zeros((N,), jnp.float32)

    agg0_k = _make_sc_aggregate(N, D, CH, with_deg=True)
    agg_k = _make_sc_aggregate(N, H, CH, with_deg=False)

    parts0, deg_parts = agg0_k(x, src2d, dst2d, z2, z1)
    deg_t = deg_parts.T

    b0r, b1r = b0.reshape(1, -1), b1.reshape(1, -1)
    g0r, be0r = gamma0.reshape(1, -1), beta0.reshape(1, -1)
    g1r, be1r = gamma1.reshape(1, -1), beta1.reshape(1, -1)

    h1 = _tc_layer(x, parts0, deg_t, W_self0, W_neigh0, b0r, g0r, be0r, True)
    (parts1,) = agg_k(h1, src2d, dst2d, z2)
    h2 = _tc_layer(h1, parts1, deg_t, W_self1, W_neigh1, b1r, g1r, be1r, True)
    (parts2,) = agg_k(h2, src2d, dst2d, z2)

    pad = H - C
    Ws2p = jnp.pad(W_self2, ((0, 0), (0, pad)))
    Wn2p = jnp.pad(W_neigh2, ((0, 0), (0, pad)))
    b2p = jnp.pad(b2, (0, pad)).reshape(1, -1)
    ones_r = jnp.ones((1, H), jnp.float32)
    zeros_r = jnp.zeros((1, H), jnp.float32)
    out = _tc_layer(h2, parts2, deg_t, Ws2p, Wn2p, b2p, ones_r, zeros_r, False)
    return out[:, :C]


# speedup vs baseline: 10.6915x; 1.5690x over previous
import functools

import jax
import jax.numpy as jnp
from jax import lax
from jax.experimental import pallas as pl
from jax.experimental.pallas import tpu as pltpu
from jax.experimental.pallas import tpu_sc as plsc

NC = 2
NS = 16
K = 125
NBUF = 4


def _make_sc_aggregate(N, F2, CH, with_deg):
    chunks = CH // NS
    half = chunks // 2
    rows_per_tile = (N // NS) // 8 * 8
    tail_rows = N - NS * rows_per_tile
    tail_base = NS * rows_per_tile
    mesh = plsc.VectorSubcoreMesh(core_axis_name="c", subcore_axis_name="s")

    out_type = [jax.ShapeDtypeStruct((NC, N, F2), jnp.float32)]
    scratch = [pltpu.VMEM((chunks, K), jnp.int32),
               pltpu.VMEM((chunks, K), jnp.int32)]
    scratch += [pltpu.VMEM((K, F2), jnp.float32) for _ in range(NBUF)]
    scratch += [pltpu.VMEM_SHARED((N, F2), jnp.float32)]
    scratch += [pltpu.SemaphoreType.DMA for _ in range(2 * NBUF)]
    if with_deg:
        out_type.append(jax.ShapeDtypeStruct((NC, N), jnp.float32))
        scratch += [
            pltpu.VMEM((128,), jnp.float32),
            pltpu.VMEM_SHARED((N,), jnp.float32),
        ]

    def _agg_body(t_hbm, src_hbm, dst_hbm, z2_hbm, out_hbm,
                  src_v, dst_v, bufs, acc, gsems, ssems,
                  z1_hbm=None, deg_hbm=None, ones_v=None, dacc=None):
        c = lax.axis_index("c")
        s = lax.axis_index("s")
        pltpu.sync_copy(z2_hbm.at[pl.ds(s * rows_per_tile, rows_per_tile)],
                        acc.at[pl.ds(s * rows_per_tile, rows_per_tile)])
        if tail_rows:
            @pl.when(s == NS - 1)
            def _():
                pltpu.sync_copy(z2_hbm.at[pl.ds(tail_base, tail_rows)],
                                acc.at[pl.ds(tail_base, tail_rows)])
        if dacc is not None:
            @pl.when(s == 0)
            def _():
                pltpu.sync_copy(z1_hbm, dacc)
            for j in range(8):
                ones_v[pl.ds(j * 16, 16)] = jnp.ones((16,), jnp.float32)
        pltpu.sync_copy(src_hbm.at[s], src_v)
        pltpu.sync_copy(dst_hbm.at[s], dst_v)
        plsc.subcore_barrier()

        def gather_issue(i, b):
            pltpu.async_copy(t_hbm.at[c].at[src_v.at[i]], bufs[b], gsems[b])

        def gather_wait(i, b):
            pltpu.make_async_copy(t_hbm.at[c].at[src_v.at[i]], bufs[b],
                                  gsems[b]).wait()

        def scatter_issue(i, b):
            pltpu.async_copy(bufs[b], acc.at[dst_v.at[i]], ssems[b], add=True)

        def scatter_wait(i, b):
            pltpu.make_async_copy(bufs[b], acc.at[dst_v.at[i]],
                                  ssems[b]).wait()

        for b in range(NBUF):
            gather_issue(b, b)

        def step(i2, carry):
            for b in range(NBUF):
                i = i2 * NBUF + b
                gather_wait(i, b)
                scatter_issue(i, b)
                if dacc is not None:
                    in_my_half = jnp.where(c == 0, i < half, i >= half)

                    @pl.when(in_my_half)
                    def _():
                        pltpu.sync_copy(ones_v.at[pl.ds(0, K)],
                                        dacc.at[dst_v.at[i]], add=True)
                pb = (b - 1) % NBUF

                @pl.when(i >= 1)
                def _():
                    scatter_wait(i - 1, pb)

                @pl.when(jnp.logical_and(i >= 1, i - 1 + NBUF < chunks))
                def _():
                    gather_issue(i - 1 + NBUF, pb)
            return carry

        lax.fori_loop(0, chunks // NBUF, step, 0)
        scatter_wait(chunks - 1, (chunks - 1) % NBUF)
        plsc.subcore_barrier()
        pltpu.sync_copy(acc.at[pl.ds(s * rows_per_tile, rows_per_tile)],
                        out_hbm.at[c, pl.ds(s * rows_per_tile, rows_per_tile)])
        if tail_rows:
            @pl.when(s == NS - 1)
            def _():
                pltpu.sync_copy(acc.at[pl.ds(tail_base, tail_rows)],
                                out_hbm.at[c, pl.ds(tail_base, tail_rows)])
        if dacc is not None:
            @pl.when(s == 0)
            def _():
                pltpu.sync_copy(dacc, deg_hbm.at[c])

    if with_deg:
        def body(t_hbm, src_hbm, dst_hbm, z2_hbm, z1_hbm, out_hbm, deg_hbm,
                 src_v, dst_v, *rest):
            bufs = rest[:NBUF]
            acc = rest[NBUF]
            gsems = rest[NBUF + 1:NBUF + 1 + NBUF]
            ssems = rest[NBUF + 1 + NBUF:NBUF + 1 + 2 * NBUF]
            ones_v, dacc = rest[-2], rest[-1]
            _agg_body(t_hbm, src_hbm, dst_hbm, z2_hbm, out_hbm,
                      src_v, dst_v, bufs, acc, gsems, ssems,
                      z1_hbm=z1_hbm, deg_hbm=deg_hbm, ones_v=ones_v,
                      dacc=dacc)
    else:
        def body(t_hbm, src_hbm, dst_hbm, z2_hbm, out_hbm,
                 src_v, dst_v, *rest):
            bufs = rest[:NBUF]
            acc = rest[NBUF]
            gsems = rest[NBUF + 1:NBUF + 1 + NBUF]
            ssems = rest[NBUF + 1 + NBUF:NBUF + 1 + 2 * NBUF]
            _agg_body(t_hbm, src_hbm, dst_hbm, z2_hbm, out_hbm,
                      src_v, dst_v, bufs, acc, gsems, ssems)

    return pl.kernel(body, out_type=tuple(out_type), mesh=mesh,
                     scratch_types=scratch,
                     compiler_params=pltpu.CompilerParams(
                         use_tc_tiling_on_sc=False))


def _tc_layer(h_st, parts, deg_t, W_self, W_neigh, b, gamma, beta, apply_ln,
              split_out):
    F2 = h_st.shape[2]
    N = h_st.shape[1]
    Fin = NC * F2
    Fout = W_self.shape[1]
    BLK = 1000
    grid = (N // BLK,)

    def body(h_ref, p_ref, d_ref, ws_ref, wn_ref, b_ref, g_ref, be_ref, o_ref):
        hb = jnp.concatenate([h_ref[0], h_ref[1]], axis=-1)
        agg = jnp.concatenate([p_ref[0], p_ref[1]], axis=-1)
        deg = jnp.sum(d_ref[...], axis=1, keepdims=True)
        inv = 1.0 / jnp.maximum(deg, 1.0)
        hn = agg * inv
        z = jnp.dot(hb, ws_ref[...], preferred_element_type=jnp.float32)
        z = z + jnp.dot(hn, wn_ref[...], preferred_element_type=jnp.float32)
        z = z + b_ref[...]
        if apply_ln:
            mu = jnp.mean(z, axis=-1, keepdims=True)
            var = jnp.mean((z - mu) ** 2, axis=-1, keepdims=True)
            z = (z - mu) * lax.rsqrt(var + 1e-5) * g_ref[...] + be_ref[...]
            z = jnp.maximum(z, 0.0)
        if split_out:
            o_ref[0] = z[:, :F2]
            o_ref[1] = z[:, F2:]
        else:
            o_ref[...] = z

    if split_out:
        out_spec = pl.BlockSpec((NC, BLK, Fout // NC), lambda i: (0, i, 0))
        out_shape = jax.ShapeDtypeStruct((NC, N, Fout // NC), jnp.float32)
    else:
        out_spec = pl.BlockSpec((BLK, Fout), lambda i: (i, 0))
        out_shape = jax.ShapeDtypeStruct((N, Fout), jnp.float32)

    return pl.pallas_call(
        body,
        grid=grid,
        in_specs=[
            pl.BlockSpec((NC, BLK, F2), lambda i: (0, i, 0)),
            pl.BlockSpec((NC, BLK, F2), lambda i: (0, i, 0)),
            pl.BlockSpec((BLK, NC), lambda i: (i, 0)),
            pl.BlockSpec((Fin, Fout), lambda i: (0, 0)),
            pl.BlockSpec((Fin, Fout), lambda i: (0, 0)),
            pl.BlockSpec((1, Fout), lambda i: (0, 0)),
            pl.BlockSpec((1, Fout), lambda i: (0, 0)),
            pl.BlockSpec((1, Fout), lambda i: (0, 0)),
        ],
        out_specs=out_spec,
        out_shape=out_shape,
    )(h_st, parts, deg_t, W_self, W_neigh, b, gamma, beta)


def kernel(x, edge_index, W_self0, W_neigh0, b0, W_self1, W_neigh1, b1,
           W_self2, W_neigh2, b2, gamma0, beta0, gamma1, beta1):
    N, D = x.shape
    E = edge_index.shape[1]
    H = W_self0.shape[1]
    C = W_self2.shape[1]
    F2 = D // NC
    CH = E // K
    assert E % K == 0 and CH % NS == 0 and (CH // NS) % NBUF == 0

    src2d = edge_index[0].astype(jnp.int32).reshape(NS, CH // NS, K)
    dst2d = edge_index[1].astype(jnp.int32).reshape(NS, CH // NS, K)
    z2 = jnp.zeros((N, F2), jnp.float32)
    z1 = jnp.zeros((N,), jnp.float32)

    agg0_k = _make_sc_aggregate(N, F2, CH, with_deg=True)
    agg_k = _make_sc_aggregate(N, F2, CH, with_deg=False)

    x_st = jnp.stack([x[:, :F2], x[:, F2:]])
    parts0, deg_parts = agg0_k(x_st, src2d, dst2d, z2, z1)
    deg_t = deg_parts.T

    b0r, b1r = b0.reshape(1, -1), b1.reshape(1, -1)
    g0r, be0r = gamma0.reshape(1, -1), beta0.reshape(1, -1)
    g1r, be1r = gamma1.reshape(1, -1), beta1.reshape(1, -1)

    h1 = _tc_layer(x_st, parts0, deg_t, W_self0, W_neigh0, b0r, g0r, be0r,
                   True, True)
    (parts1,) = agg_k(h1, src2d, dst2d, z2)
    h2 = _tc_layer(h1, parts1, deg_t, W_self1, W_neigh1, b1r, g1r, be1r,
                   True, True)
    (parts2,) = agg_k(h2, src2d, dst2d, z2)

    pad = H - C
    Ws2p = jnp.pad(W_self2, ((0, 0), (0, pad)))
    Wn2p = jnp.pad(W_neigh2, ((0, 0), (0, pad)))
    b2p = jnp.pad(b2, (0, pad)).reshape(1, -1)
    ones_r = jnp.ones((1, H), jnp.float32)
    zeros_r = jnp.zeros((1, H), jnp.float32)
    out = _tc_layer(h2, parts2, deg_t, Ws2p, Wn2p, b2p, ones_r, zeros_r,
                    False, False)
    return out[:, :C]


# speedup vs baseline: 15.1959x; 1.4213x over previous
import functools

import jax
import jax.numpy as jnp
from jax import lax
from jax.experimental import pallas as pl
from jax.experimental.pallas import tpu as pltpu
from jax.experimental.pallas import tpu_sc as plsc

NC = 2
NS = 16
K = 125
NBUF = 5


def _make_sc_aggregate(N, F2, CH, with_deg, strip_mode):
    chunks = CH // NS if strip_mode else CH // (NC * NS)
    half = chunks // 2
    rows_per_tile = (N // NS) // 8 * 8
    tail_rows = N - NS * rows_per_tile
    tail_base = NS * rows_per_tile
    mesh = plsc.VectorSubcoreMesh(core_axis_name="c", subcore_axis_name="s")

    out_type = [jax.ShapeDtypeStruct((N, NC * F2), jnp.float32)]
    scratch = [pltpu.VMEM((chunks, K), jnp.int32),
               pltpu.VMEM((chunks, K), jnp.int32)]
    scratch += [pltpu.VMEM((K, F2), jnp.float32) for _ in range(NBUF)]
    scratch += [pltpu.VMEM_SHARED((N, F2), jnp.float32)]
    scratch += [pltpu.SemaphoreType.DMA for _ in range(2 * NBUF)]
    if with_deg:
        out_type.append(jax.ShapeDtypeStruct((NC, N), jnp.float32))
        scratch += [
            pltpu.VMEM((128,), jnp.float32),
            pltpu.VMEM_SHARED((N,), jnp.float32),
        ]

    def _agg_body(t_hbm, src_hbm, dst_hbm, z2_hbm, out_hbm,
                  src_v, dst_v, bufs, acc, gsems, ssems,
                  z1_hbm=None, deg_hbm=None, ones_v=None, dacc=None):
        c = lax.axis_index("c")
        s = lax.axis_index("s")
        strip = c if strip_mode else 0
        chunk0 = (s if strip_mode else c * NS + s) * chunks
        pltpu.sync_copy(z2_hbm.at[pl.ds(s * rows_per_tile, rows_per_tile)],
                        acc.at[pl.ds(s * rows_per_tile, rows_per_tile)])
        if tail_rows:
            @pl.when(s == NS - 1)
            def _():
                pltpu.sync_copy(z2_hbm.at[pl.ds(tail_base, tail_rows)],
                                acc.at[pl.ds(tail_base, tail_rows)])
        if dacc is not None:
            @pl.when(s == 0)
            def _():
                pltpu.sync_copy(z1_hbm, dacc)
            for j in range(8):
                ones_v[pl.ds(j * 16, 16)] = jnp.ones((16,), jnp.float32)
        pltpu.sync_copy(src_hbm.at[strip, pl.ds(chunk0, chunks)], src_v)
        pltpu.sync_copy(dst_hbm.at[pl.ds(chunk0, chunks)], dst_v)
        plsc.subcore_barrier()

        def gather_issue(i, b):
            pltpu.async_copy(t_hbm.at[src_v.at[i]], bufs[b], gsems[b])

        def gather_wait(i, b):
            pltpu.make_async_copy(t_hbm.at[src_v.at[i]], bufs[b],
                                  gsems[b]).wait()

        def scatter_issue(i, b):
            pltpu.async_copy(bufs[b], acc.at[dst_v.at[i]], ssems[b], add=True)

        def scatter_wait(i, b):
            pltpu.make_async_copy(bufs[b], acc.at[dst_v.at[i]],
                                  ssems[b]).wait()

        for b in range(NBUF):
            gather_issue(b, b)

        def step(i2, carry):
            for b in range(NBUF):
                i = i2 * NBUF + b
                gather_wait(i, b)
                scatter_issue(i, b)
                if dacc is not None:
                    in_my_half = jnp.where(c == 0, i < half, i >= half)

                    @pl.when(in_my_half)
                    def _():
                        pltpu.sync_copy(ones_v.at[pl.ds(0, K)],
                                        dacc.at[dst_v.at[i]], add=True)
                pb = (b - 1) % NBUF

                @pl.when(i >= 1)
                def _():
                    scatter_wait(i - 1, pb)

                @pl.when(jnp.logical_and(i >= 1, i - 1 + NBUF < chunks))
                def _():
                    gather_issue(i - 1 + NBUF, pb)
            return carry

        lax.fori_loop(0, chunks // NBUF, step, 0)
        scatter_wait(chunks - 1, (chunks - 1) % NBUF)
        plsc.subcore_barrier()
        out_cols = out_hbm.at[:, pl.ds(c * F2, F2)]
        pltpu.sync_copy(acc.at[pl.ds(s * rows_per_tile, rows_per_tile)],
                        out_cols.at[pl.ds(s * rows_per_tile, rows_per_tile)])
        if tail_rows:
            @pl.when(s == NS - 1)
            def _():
                pltpu.sync_copy(acc.at[pl.ds(tail_base, tail_rows)],
                                out_cols.at[pl.ds(tail_base, tail_rows)])
        if dacc is not None:
            @pl.when(s == 0)
            def _():
                pltpu.sync_copy(dacc, deg_hbm.at[c])

    if with_deg:
        def body(t_hbm, src_hbm, dst_hbm, z2_hbm, z1_hbm, out_hbm, deg_hbm,
                 src_v, dst_v, *rest):
            bufs = rest[:NBUF]
            acc = rest[NBUF]
            gsems = rest[NBUF + 1:NBUF + 1 + NBUF]
            ssems = rest[NBUF + 1 + NBUF:NBUF + 1 + 2 * NBUF]
            ones_v, dacc = rest[-2], rest[-1]
            _agg_body(t_hbm, src_hbm, dst_hbm, z2_hbm, out_hbm,
                      src_v, dst_v, bufs, acc, gsems, ssems,
                      z1_hbm=z1_hbm, deg_hbm=deg_hbm, ones_v=ones_v,
                      dacc=dacc)
    else:
        def body(t_hbm, src_hbm, dst_hbm, z2_hbm, out_hbm,
                 src_v, dst_v, *rest):
            bufs = rest[:NBUF]
            acc = rest[NBUF]
            gsems = rest[NBUF + 1:NBUF + 1 + NBUF]
            ssems = rest[NBUF + 1 + NBUF:NBUF + 1 + 2 * NBUF]
            _agg_body(t_hbm, src_hbm, dst_hbm, z2_hbm, out_hbm,
                      src_v, dst_v, bufs, acc, gsems, ssems)

    return pl.kernel(body, out_type=tuple(out_type), mesh=mesh,
                     scratch_types=scratch,
                     compiler_params=pltpu.CompilerParams(
                         use_tc_tiling_on_sc=False))


def _tc_layer(h, agg, deg2, W_self, W_neigh, b, gamma, beta, apply_ln,
              sum_halves, W_pre):
    N, Fin = h.shape
    Fout = W_self.shape[1]
    BLK = 2000
    grid = (N // BLK,)
    F2 = Fin // 2

    def body(h_ref, p_ref, d_ref, ws_ref, wn_ref, b_ref, g_ref, be_ref,
             wp_ref, *outs):
        hb = h_ref[...]
        deg = jnp.sum(d_ref[...], axis=1, keepdims=True)
        inv = 1.0 / jnp.maximum(deg, 1.0)
        p = p_ref[...]
        z = jnp.dot(hb, ws_ref[...], preferred_element_type=jnp.float32)
        if sum_halves:
            agg = p[:, :F2] + p[:, F2:]
            zeros = jnp.zeros_like(agg)
            z = z + jnp.concatenate([agg, zeros], axis=-1) * inv
        else:
            z = z + jnp.dot(p * inv, wn_ref[...],
                            preferred_element_type=jnp.float32)
        z = z + b_ref[...]
        if apply_ln:
            mu = jnp.mean(z, axis=-1, keepdims=True)
            var = jnp.mean((z - mu) ** 2, axis=-1, keepdims=True)
            z = (z - mu) * lax.rsqrt(var + 1e-5) * g_ref[...] + be_ref[...]
            z = jnp.maximum(z, 0.0)
        outs[0][...] = z
        if len(outs) > 1:
            outs[1][...] = jnp.dot(z, wp_ref[...],
                                   preferred_element_type=jnp.float32)

    n_out = 2 if W_pre is not None else 1
    wp = W_pre if W_pre is not None else jnp.zeros((Fout, 8), jnp.float32)
    out_shape = [jax.ShapeDtypeStruct((N, Fout), jnp.float32)]
    out_specs = [pl.BlockSpec((BLK, Fout), lambda i: (i, 0))]
    if n_out == 2:
        out_shape.append(jax.ShapeDtypeStruct((N, wp.shape[1]), jnp.float32))
        out_specs.append(pl.BlockSpec((BLK, wp.shape[1]), lambda i: (i, 0)))

    res = pl.pallas_call(
        body,
        grid=grid,
        in_specs=[
            pl.BlockSpec((BLK, Fin), lambda i: (i, 0)),
            pl.BlockSpec((BLK, Fin), lambda i: (i, 0)),
            pl.BlockSpec((BLK, NC), lambda i: (i, 0)),
            pl.BlockSpec((Fin, Fout), lambda i: (0, 0)),
            pl.BlockSpec((Fin, Fout), lambda i: (0, 0)),
            pl.BlockSpec((1, Fout), lambda i: (0, 0)),
            pl.BlockSpec((1, Fout), lambda i: (0, 0)),
            pl.BlockSpec((1, Fout), lambda i: (0, 0)),
            pl.BlockSpec(wp.shape, lambda i: (0, 0)),
        ],
        out_specs=out_specs,
        out_shape=out_shape,
    )(h, agg, deg2, W_self, W_neigh, b, gamma, beta, wp)
    return res


def kernel(x, edge_index, W_self0, W_neigh0, b0, W_self1, W_neigh1, b1,
           W_self2, W_neigh2, b2, gamma0, beta0, gamma1, beta1):
    N, D = x.shape
    E = edge_index.shape[1]
    H = W_self0.shape[1]
    C = W_self2.shape[1]
    F2 = D // NC
    CH = E // K
    assert E % K == 0 and CH % (NC * NS) == 0

    src1 = edge_index[0].astype(jnp.int32).reshape(CH, K)
    src2d = jnp.stack([2 * src1, 2 * src1 + 1])
    dst2d = edge_index[1].astype(jnp.int32).reshape(CH, K)
    z2 = jnp.zeros((N, F2), jnp.float32)
    z1 = jnp.zeros((N,), jnp.float32)

    agg0_k = _make_sc_aggregate(N, F2, CH, with_deg=True, strip_mode=True)
    agg_k = _make_sc_aggregate(N, F2, CH, with_deg=False, strip_mode=True)
    agg2_k = _make_sc_aggregate(N, F2, CH, with_deg=False, strip_mode=False)

    parts0, deg_parts = agg0_k(x.reshape(NC * N, F2), src2d, dst2d, z2, z1)
    deg2 = deg_parts.T

    b0r, b1r = b0.reshape(1, -1), b1.reshape(1, -1)
    g0r, be0r = gamma0.reshape(1, -1), beta0.reshape(1, -1)
    g1r, be1r = gamma1.reshape(1, -1), beta1.reshape(1, -1)

    pad = H - C
    Ws2p = jnp.pad(W_self2, ((0, 0), (0, pad)))
    Wn2p = jnp.pad(W_neigh2, ((0, 0), (0, pad)))
    b2p = jnp.pad(b2, (0, pad)).reshape(1, -1)
    ones_r = jnp.ones((1, H), jnp.float32)
    zeros_r = jnp.zeros((1, H), jnp.float32)

    (h1,) = _tc_layer(x, parts0, deg2, W_self0, W_neigh0, b0r, g0r, be0r,
                      True, False, None)
    (parts1,) = agg_k(h1.reshape(NC * N, F2), src2d, dst2d, z2)
    h2, t2 = _tc_layer(h1, parts1, deg2, W_self1, W_neigh1, b1r, g1r,
                       be1r, True, False, Wn2p)
    (parts2,) = agg2_k(t2.reshape(NC * N, F2), src2d, dst2d, z2)
    (out,) = _tc_layer(h2, parts2, deg2, Ws2p, Wn2p, b2p, ones_r, zeros_r,
                       False, True, None)
    return out[:, :C]
